# baseline (speedup 1.0000x reference)
import functools

import jax
import jax.numpy as jnp
from jax import lax
from jax.experimental import pallas as pl
from jax.experimental.pallas import tpu as pltpu
from jax.experimental.pallas import tpu_sc as plsc

VOCAB = 100001
D = 128
B = 4096
L = 200

NUM_CORES = 2
NUM_SUBCORES = 16
NW = NUM_CORES * NUM_SUBCORES
S_PER_W = B // NW
LANES = 16
DW = D // 2
NM = DW // LANES
NBUF = 4
UNROLL = 4

SPLIT = 128


def _sc_body(idx_hbm, table_hbm, out_hbm, idx_v, rows, out_v, sems):
    wid = lax.axis_index("s") * NUM_CORES + lax.axis_index("c")
    base = wid * (S_PER_W * L)

    pltpu.sync_copy(idx_hbm.at[pl.ds(base, S_PER_W * L)], idx_v)

    def start_gather(i, b):
        off = pl.multiple_of(i * L, 8)
        pltpu.async_copy(
            table_hbm.at[idx_v.at[pl.ds(off, SPLIT)]],
            rows[b].at[pl.ds(0, SPLIT)], sems[b])
        pltpu.async_copy(
            table_hbm.at[idx_v.at[pl.ds(off + SPLIT, L - SPLIT)]],
            rows[b].at[pl.ds(SPLIT, L - SPLIT)], sems[b])

    def wait_gather(b):
        pltpu.make_async_copy(table_hbm.at[pl.ds(0, L)], rows[b], sems[b]).wait()

    def accumulate(b, s):
        def add_row(r, acc):
            for m in range(NM):
                v = rows[b][r, pl.ds(m * LANES, LANES)]
                lo = lax.bitcast_convert_type(v << 16, jnp.float32)
                hi = lax.bitcast_convert_type(v, jnp.float32)
                acc = (acc[:2 * m] + (acc[2 * m] + lo, acc[2 * m + 1] + hi)
                       + acc[2 * m + 2:])
            return acc

        def body(i, acc):
            r = pl.multiple_of(i * UNROLL, UNROLL)
            for u in range(UNROLL):
                acc = add_row(r + u, acc)
            return acc

        acc0 = tuple(jnp.zeros((LANES,), jnp.float32) for _ in range(2 * NM))
        acc = lax.fori_loop(0, L // UNROLL, body, acc0)
        inv = jnp.float32(1.0 / L)
        for m in range(NM):
            out_v[s, pl.ds(2 * m * LANES, LANES)] = acc[2 * m] * inv
            out_v[s, pl.ds((2 * m + 1) * LANES, LANES)] = acc[2 * m + 1] * inv

    for b in range(NBUF):
        start_gather(b, b)

    def outer(k, carry):
        s0 = k * NBUF
        for b in range(NBUF):
            wait_gather(b)
            accumulate(b, s0 + b)

            @pl.when(s0 + b + NBUF < S_PER_W)
            def _():
                start_gather(s0 + b + NBUF, b)

        return carry

    lax.fori_loop(0, S_PER_W // NBUF, outer, 0)

    pltpu.sync_copy(out_v, out_hbm.at[pl.ds(wid * S_PER_W, S_PER_W)])


@functools.partial(
    pl.kernel,
    out_type=jax.ShapeDtypeStruct((B, D), jnp.float32),
    mesh=plsc.VectorSubcoreMesh(core_axis_name="c", subcore_axis_name="s"),
    compiler_params=pltpu.CompilerParams(use_tc_tiling_on_sc=False),
    scratch_types=[
        pltpu.VMEM((S_PER_W * L,), jnp.int32),
        [pltpu.VMEM((L, DW), jnp.int32) for _ in range(NBUF)],
        pltpu.VMEM((S_PER_W, D), jnp.float32),
        [pltpu.SemaphoreType.DMA for _ in range(NBUF)],
    ],
)
def _sc_avg(idx_hbm, table_hbm, out_hbm, idx_v, rows, out_v, sems):
    _sc_body(idx_hbm, table_hbm, out_hbm, idx_v, rows, out_v, sems)


def kernel(indices, table):
    idx_flat = indices.astype(jnp.int32).reshape(B * L)
    t4 = table.astype(jnp.bfloat16).reshape(VOCAB, NM, 2, LANES)
    packed = lax.bitcast_convert_type(
        t4.transpose(0, 1, 3, 2).reshape(VOCAB, DW, 2), jnp.int32)
    return _sc_avg(idx_flat, packed)


# speedup vs baseline: 1.5881x; 1.5881x over previous
import functools

import jax
import jax.numpy as jnp
from jax import lax
from jax.experimental import pallas as pl
from jax.experimental.pallas import tpu as pltpu
from jax.experimental.pallas import tpu_sc as plsc

VOCAB = 100001
D = 128
B = 4096
L = 200

NUM_CORES = 2
NUM_SUBCORES = 16
NW = NUM_CORES * NUM_SUBCORES
S_PER_W = B // NW
LANES = 16
DW = D // 2
NM = DW // LANES
NBUF = 4
UNROLL = 4

SPLIT = 128

ROWS_PER_W = 3126
CHUNK = 64
NCHUNK = 50


def _mesh():
    return plsc.VectorSubcoreMesh(core_axis_name="c", subcore_axis_name="s")


def _wid():
    return lax.axis_index("s") * NUM_CORES + lax.axis_index("c")


def _pack_body(table_hbm, packed_hbm, ibufs, obufs, isems, osems):
    wid = _wid()
    w_start = wid * ROWS_PER_W
    w_last = jnp.minimum(w_start + ROWS_PER_W, VOCAB) - CHUNK

    def chunk_start(c):
        return jnp.minimum(w_start + c * CHUNK, w_last)

    def start_in(c, b):
        pltpu.async_copy(
            table_hbm.at[pl.ds(chunk_start(c), CHUNK)], ibufs[b], isems[b])

    def wait_in(b):
        pltpu.make_async_copy(
            table_hbm.at[pl.ds(0, CHUNK)], ibufs[b], isems[b]).wait()

    def wait_out(b):
        pltpu.make_async_copy(
            table_hbm.at[pl.ds(0, CHUNK)], obufs[b], osems[b]).wait()

    def pack_chunk(b):
        half = jnp.int32(0x8000)
        mask = jnp.int32(-65536)

        def body(i, carry):
            r = pl.multiple_of(i * 2, 2)
            for u in range(2):
                for m in range(NM):
                    a = lax.bitcast_convert_type(
                        ibufs[b][r + u, pl.ds(32 * m, LANES)], jnp.int32)
                    c = lax.bitcast_convert_type(
                        ibufs[b][r + u, pl.ds(32 * m + LANES, LANES)], jnp.int32)
                    lo = lax.shift_right_logical(a + half, 16)
                    hi = (c + half) & mask
                    obufs[b][r + u, pl.ds(m * LANES, LANES)] = lo | hi
            return carry

        lax.fori_loop(0, CHUNK // 2, body, 0)

    def start_out(c, b):
        pltpu.async_copy(
            obufs[b], packed_hbm.at[pl.ds(chunk_start(c), CHUNK)], osems[b])

    start_in(0, 0)

    def outer(k, carry):
        c0 = k * 2
        start_in(c0 + 1, 1)
        wait_in(0)

        @pl.when(k > 0)
        def _():
            wait_out(0)

        pack_chunk(0)
        start_out(c0, 0)

        @pl.when(c0 + 2 < NCHUNK)
        def _():
            start_in(c0 + 2, 0)

        wait_in(1)

        @pl.when(k > 0)
        def _():
            wait_out(1)

        pack_chunk(1)
        start_out(c0 + 1, 1)
        return carry

    lax.fori_loop(0, NCHUNK // 2, outer, 0)
    wait_out(0)
    wait_out(1)


@functools.partial(
    pl.kernel,
    out_type=jax.ShapeDtypeStruct((VOCAB, DW), jnp.int32),
    mesh=_mesh(),
    compiler_params=pltpu.CompilerParams(use_tc_tiling_on_sc=False),
    scratch_types=[
        [pltpu.VMEM((CHUNK, D), jnp.float32) for _ in range(2)],
        [pltpu.VMEM((CHUNK, DW), jnp.int32) for _ in range(2)],
        [pltpu.SemaphoreType.DMA for _ in range(2)],
        [pltpu.SemaphoreType.DMA for _ in range(2)],
    ],
)
def _sc_pack(table_hbm, packed_hbm, ibufs, obufs, isems, osems):
    _pack_body(table_hbm, packed_hbm, ibufs, obufs, isems, osems)


def _avg_body(idx_hbm, table_hbm, out_hbm, idx_v, rows, out_v, sems):
    wid = _wid()
    base = wid * (S_PER_W * L)

    pltpu.sync_copy(idx_hbm.at[pl.ds(base, S_PER_W * L)], idx_v)

    def start_gather(i, b):
        off = pl.multiple_of(i * L, 8)
        pltpu.async_copy(
            table_hbm.at[idx_v.at[pl.ds(off, SPLIT)]],
            rows[b].at[pl.ds(0, SPLIT)], sems[b])
        pltpu.async_copy(
            table_hbm.at[idx_v.at[pl.ds(off + SPLIT, L - SPLIT)]],
            rows[b].at[pl.ds(SPLIT, L - SPLIT)], sems[b])

    def wait_gather(b):
        pltpu.make_async_copy(table_hbm.at[pl.ds(0, L)], rows[b], sems[b]).wait()

    def accumulate(b, s):
        def add_row(r, acc):
            for m in range(NM):
                v = rows[b][r, pl.ds(m * LANES, LANES)]
                lo = lax.bitcast_convert_type(v << 16, jnp.float32)
                hi = lax.bitcast_convert_type(v, jnp.float32)
                acc = (acc[:2 * m] + (acc[2 * m] + lo, acc[2 * m + 1] + hi)
                       + acc[2 * m + 2:])
            return acc

        def body(i, acc):
            r = pl.multiple_of(i * UNROLL, UNROLL)
            for u in range(UNROLL):
                acc = add_row(r + u, acc)
            return acc

        acc0 = tuple(jnp.zeros((LANES,), jnp.float32) for _ in range(2 * NM))
        acc = lax.fori_loop(0, L // UNROLL, body, acc0)
        inv = jnp.float32(1.0 / L)
        for m in range(NM):
            out_v[s, pl.ds(2 * m * LANES, LANES)] = acc[2 * m] * inv
            out_v[s, pl.ds((2 * m + 1) * LANES, LANES)] = acc[2 * m + 1] * inv

    for b in range(NBUF):
        start_gather(b, b)

    def outer(k, carry):
        s0 = k * NBUF
        for b in range(NBUF):
            wait_gather(b)
            accumulate(b, s0 + b)

            @pl.when(s0 + b + NBUF < S_PER_W)
            def _():
                start_gather(s0 + b + NBUF, b)

        return carry

    lax.fori_loop(0, S_PER_W // NBUF, outer, 0)

    pltpu.sync_copy(out_v, out_hbm.at[pl.ds(wid * S_PER_W, S_PER_W)])


@functools.partial(
    pl.kernel,
    out_type=jax.ShapeDtypeStruct((B, D), jnp.float32),
    mesh=_mesh(),
    compiler_params=pltpu.CompilerParams(use_tc_tiling_on_sc=False),
    scratch_types=[
        pltpu.VMEM((S_PER_W * L,), jnp.int32),
        [pltpu.VMEM((L, DW), jnp.int32) for _ in range(NBUF)],
        pltpu.VMEM((S_PER_W, D), jnp.float32),
        [pltpu.SemaphoreType.DMA for _ in range(NBUF)],
    ],
)
def _sc_avg(idx_hbm, table_hbm, out_hbm, idx_v, rows, out_v, sems):
    _avg_body(idx_hbm, table_hbm, out_hbm, idx_v, rows, out_v, sems)


def kernel(indices, table):
    idx_flat = indices.astype(jnp.int32).reshape(B * L)
    packed = _sc_pack(table)
    return _sc_avg(idx_flat, packed)
